# baseline (speedup 1.0000x reference)
import functools

import jax
import jax.numpy as jnp
from jax import lax
from jax.experimental import pallas as pl
from jax.experimental.pallas import tpu as pltpu
from jax.experimental.pallas import tpu_sc as plsc

_B = 4
_NP = 8192
_M = 64
_NW = 32
_PPW = (_B * _NP) // _NW
_GRP = _PPW // 16
_NOHIT = 4 * _M


def _sc_body(xs_hbm, ys_hbm, zs_hbm, par_hbm, rows_hbm,
             lbl_hbm, fgbox_hbm, idx_hbm,
             xs_v, ys_v, zs_v, par_v, rows_v, lbl_v, fgb_v, idx_v):
    cid = lax.axis_index("c")
    sid = lax.axis_index("s")
    wid = sid * 2 + cid
    base = wid * _PPW
    bidx = wid // (_NP // _PPW)

    pltpu.sync_copy(xs_hbm.at[pl.ds(base, _PPW)], xs_v)
    pltpu.sync_copy(ys_hbm.at[pl.ds(base, _PPW)], ys_v)
    pltpu.sync_copy(zs_hbm.at[pl.ds(base, _PPW)], zs_v)
    pltpu.sync_copy(par_hbm.at[bidx], par_v)
    pltpu.sync_copy(rows_hbm.at[bidx], rows_v)

    lanes = lax.iota(jnp.int32, 16)

    def group_body(g, _):
        off = g * 16
        px = xs_v[pl.ds(off, 16)]
        py = ys_v[pl.ds(off, 16)]
        pz = zs_v[pl.ds(off, 16)]

        def box_body(m, carry):
            enc, anyx = carry
            mf = jnp.full((16,), m, jnp.int32)

            def par(k):
                return plsc.load_gather(par_v, [mf + (k * _M)])

            sx = px - par(0)
            sy = py - par(1)
            sz = pz - par(2)
            c = par(3)
            s = par(4)
            lx = jnp.abs(sx * c - sy * s)
            ly = jnp.abs(sx * s + sy * c)
            az = jnp.abs(sz)
            in_gt = (lx <= par(5)) & (ly <= par(6)) & (az <= par(7))
            in_ex = (lx <= par(8)) & (ly <= par(9)) & (az <= par(10))
            encv = mf * 4 + par(11).astype(jnp.int32)
            enc = jnp.where(in_gt, jnp.minimum(enc, encv), enc)
            anyx = anyx | jnp.where(in_ex, 1, 0)
            return enc, anyx

        enc0 = jnp.full((16,), _NOHIT, jnp.int32)
        anyx0 = jnp.zeros((16,), jnp.int32)
        enc, anyx = lax.fori_loop(0, _M, box_body, (enc0, anyx0))

        fg = enc < _NOHIT
        ig = jnp.logical_xor(fg, anyx > 0)
        fst = enc >> 2
        cls = enc & 3
        idxv = jnp.where(fg, fst, -1)
        clamped = jnp.maximum(idxv, 0)
        lblv = jnp.where(fg, cls, jnp.where(ig, -1, 0))

        lbl_v[pl.ds(off, 16)] = lblv
        idx_v[pl.ds(off, 16)] = idxv
        pids = off + lanes
        for j in range(8):
            vals = plsc.load_gather(rows_v, [clamped * 8 + j])
            plsc.store_scatter(fgb_v, [pids * 8 + j], vals)
        return 0

    lax.fori_loop(0, _GRP, group_body, 0)

    pltpu.sync_copy(lbl_v, lbl_hbm.at[pl.ds(base, _PPW)])
    pltpu.sync_copy(idx_v, idx_hbm.at[pl.ds(base, _PPW)])
    pltpu.sync_copy(fgb_v, fgbox_hbm.at[pl.ds(base * 8, _PPW * 8)])


def kernel(points, gt_boxes, extend_gt_boxes):
    n = points.shape[0]
    xs = points[:, 1]
    ys = points[:, 2]
    zs = points[:, 3]
    h = gt_boxes[:, :, 6]
    par = jnp.stack(
        [
            gt_boxes[:, :, 0],
            gt_boxes[:, :, 1],
            gt_boxes[:, :, 2],
            jnp.cos(-h),
            jnp.sin(-h),
            gt_boxes[:, :, 3] / 2.0,
            gt_boxes[:, :, 4] / 2.0,
            gt_boxes[:, :, 5] / 2.0,
            extend_gt_boxes[:, :, 3] / 2.0,
            extend_gt_boxes[:, :, 4] / 2.0,
            extend_gt_boxes[:, :, 5] / 2.0,
            gt_boxes[:, :, 7],
        ],
        axis=1,
    )

    mesh = plsc.VectorSubcoreMesh(core_axis_name="c", subcore_axis_name="s")
    run = functools.partial(
        pl.kernel,
        mesh=mesh,
        compiler_params=pltpu.CompilerParams(needs_layout_passes=False),
        out_type=[
            jax.ShapeDtypeStruct((n,), jnp.int32),
            jax.ShapeDtypeStruct((n * 8,), jnp.float32),
            jax.ShapeDtypeStruct((n,), jnp.int32),
        ],
        scratch_types=[
            pltpu.VMEM((_PPW,), jnp.float32),
            pltpu.VMEM((_PPW,), jnp.float32),
            pltpu.VMEM((_PPW,), jnp.float32),
            pltpu.VMEM((12 * _M,), jnp.float32),
            pltpu.VMEM((_M * 8,), jnp.float32),
            pltpu.VMEM((_PPW,), jnp.int32),
            pltpu.VMEM((_PPW * 8,), jnp.float32),
            pltpu.VMEM((_PPW,), jnp.int32),
        ],
    )(_sc_body)
    par_flat = par.reshape(_B, 12 * _M)
    rows_flat = gt_boxes.reshape(_B, _M * 8)
    lbl, fgbox, idx = run(xs, ys, zs, par_flat, rows_flat)
    return lbl, fgbox.reshape(n, 8), idx
